# Initial kernel scaffold; baseline (speedup 1.0000x reference)
import jax
import jax.numpy as jnp
from jax.experimental import pallas as pl


def kernel(edge_index, edge_type, weight1, root1, bias1, weight2, root2, bias2):
    raise NotImplementedError("write your pallas kernel here")


# speedup vs baseline: 1.9232x; 1.9232x over previous
import functools

import jax
import jax.numpy as jnp
from jax import lax
from jax.experimental import pallas as pl
from jax.experimental.pallas import tpu as pltpu
from jax.experimental.pallas import tpu_sc as plsc

N = 10000
R = 16
H = 256
L = 128
E = 160000

NC = 2
NS = 16
LN = 16

HH = H // NC
CHUNK = 128
SEC = 2048

EPS_A = E // NS
NCH_A = -(-EPS_A // CHUNK)
PAD_A = NCH_A * CHUNK
SECTIONS_A = [(0, 16, 2048), (2048, 16, 2048), (4096, 16, 2048),
              (6144, 16, 2048), (8192, 15, 1808)]
WSTRIDE = 4 * PAD_A

EPW_C = E // (NC * NS)
NCH_C = -(-EPW_C // CHUNK)
PAD_C = NCH_C * CHUNK

RB = 624
RLAST = N - RB * (NS - 1)

_mesh = plsc.VectorSubcoreMesh(core_axis_name="c", subcore_axis_name="s")


def _zero_fill_1d(ref, n):
    z = jnp.zeros((LN,), jnp.float32)

    def body(i, _):
        ref[pl.ds(i * LN, LN)] = z
        return 0

    lax.fori_loop(0, n // LN, body, 0)


def _zero_fill_2d(ref, rows, cols):
    z = jnp.zeros((LN,), jnp.float32)

    def body(i, _):
        r = i // (cols // LN)
        c = i % (cols // LN)
        ref[r, pl.ds(c * LN, LN)] = z
        return 0

    lax.fori_loop(0, rows * (cols // LN), body, 0)


def _fill_tail_i32(ref, start, n, value):
    v = jnp.full((LN,), value, jnp.int32)
    for k in range(n // LN):
        ref[pl.ds(start + k * LN, LN)] = v


def _stage_section(srce_hbm, dste_hbm, et_hbm, src_v, dst_v, et_v,
                   ebase, sec_base, nreal):
    if nreal < SEC:
        _fill_tail_i32(src_v, nreal, SEC - nreal, 0)
        _fill_tail_i32(dst_v, nreal, SEC - nreal, N)
        _fill_tail_i32(et_v, nreal, SEC - nreal, 0)
    off = pl.multiple_of(ebase + sec_base, 8)
    pltpu.sync_copy(srce_hbm.at[pl.ds(off, nreal)], src_v.at[pl.ds(0, nreal)])
    pltpu.sync_copy(dste_hbm.at[pl.ds(off, nreal)], dst_v.at[pl.ds(0, nreal)])
    pltpu.sync_copy(et_hbm.at[pl.ds(off, nreal)], et_v.at[pl.ds(0, nreal)])


def _agg_zero(agg_sh, zsrc_v, sid):
    @pl.when(sid < NS - 1)
    def _():
        base = pl.multiple_of(sid * RB, 8)
        for k in range(4):
            pltpu.sync_copy(zsrc_v.at[pl.ds(0, 128)],
                            agg_sh.at[pl.ds(base + k * 128, 128)])
        pltpu.sync_copy(zsrc_v.at[pl.ds(0, 112)],
                        agg_sh.at[pl.ds(base + 512, 112)])

    @pl.when(sid == NS - 1)
    def _():
        for k in range(5):
            pltpu.sync_copy(zsrc_v.at[pl.ds(0, 128)],
                            agg_sh.at[pl.ds(RB * (NS - 1) + k * 128, 128)])
        pltpu.sync_copy(zsrc_v.at[pl.ds(0, 8)], agg_sh.at[pl.ds(N, 8)])


def _agg_writeback(agg_sh, agg_out, cid, sid):
    @pl.when(sid < NS - 1)
    def _():
        base = pl.multiple_of(sid * RB, 8)
        pltpu.sync_copy(agg_sh.at[pl.ds(base, RB)],
                        agg_out.at[cid, pl.ds(base, RB)])

    @pl.when(sid == NS - 1)
    def _():
        pltpu.sync_copy(agg_sh.at[pl.ds(RB * (NS - 1), RLAST)],
                        agg_out.at[cid, pl.ds(RB * (NS - 1), RLAST)])


def _sc_kernel_a(srce_hbm, dste_hbm, et_hbm, w1_hbm, agg_out, w_out,
                 cnt_sh, agg_sh, src_v, dst_v, et_v, seg_v, g1_v, dstc_v,
                 ones_v, cw_v, wrow_v, rows_v, z1_v):
    cid = lax.axis_index("c")
    sid = lax.axis_index("s")
    ebase = sid * EPS_A

    _zero_fill_2d(rows_v, CHUNK, HH)
    _zero_fill_1d(z1_v, 2000)
    one = jnp.ones((LN,), jnp.float32)
    for k in range(CHUNK // LN):
        ones_v[pl.ds(k * LN, LN)] = one

    for k in range(5):
        pltpu.sync_copy(z1_v.at[pl.ds(0, 2000)],
                        cnt_sh.at[pl.ds(sid * EPS_A + k * 2000, 2000)])
    _agg_zero(agg_sh, rows_v, sid)

    @pl.when(sid == 0)
    def _():
        pltpu.sync_copy(z1_v.at[pl.ds(0, LN)], cnt_sh.at[pl.ds(N * R, LN)])

    plsc.subcore_barrier()

    for sec_base, nch, nreal in SECTIONS_A:
        _stage_section(srce_hbm, dste_hbm, et_hbm, src_v, dst_v, et_v,
                       ebase, sec_base, nreal)

        def count_body(g, _):
            for i in range(CHUNK // LN):
                sl = pl.ds(g * CHUNK + i * LN, LN)
                seg_v[pl.ds(i * LN, LN)] = dst_v[sl] * R + et_v[sl]
            pltpu.sync_copy(ones_v, cnt_sh.at[seg_v], add=True)
            return 0

        lax.fori_loop(0, nch, count_body, 0)

    plsc.subcore_barrier()

    for sec_base, nch, nreal in SECTIONS_A:
        _stage_section(srce_hbm, dste_hbm, et_hbm, src_v, dst_v, et_v,
                       ebase, sec_base, nreal)

        def main_body(g, _):
            for i in range(CHUNK // LN):
                sl = pl.ds(g * CHUNK + i * LN, LN)
                s16 = src_v[sl]
                d16 = dst_v[sl]
                t16 = et_v[sl]
                seg_v[pl.ds(i * LN, LN)] = d16 * R + t16
                g1_v[pl.ds(i * LN, LN)] = (t16 * N + s16) * 2 + cid
                dstc_v[pl.ds(i * LN, LN)] = d16
            pltpu.sync_copy(cnt_sh.at[seg_v], cw_v.at[pl.ds(0, CHUNK)])
            pltpu.sync_copy(w1_hbm.at[g1_v], rows_v)
            for i in range(CHUNK // LN):
                sl = pl.ds(i * LN, LN)
                w16 = 1.0 / cw_v[sl]
                cw_v[sl] = w16
                wrow_v[pl.ds(g * CHUNK + i * LN, LN)] = w16

            def scale_body(j, _):
                wj = cw_v[pl.ds(j, LN)][0]
                for hh in range(HH // LN):
                    sl2 = pl.ds(hh * LN, LN)
                    rows_v[j, sl2] = rows_v[j, sl2] * wj
                return 0

            lax.fori_loop(0, CHUNK, scale_body, 0)
            pltpu.sync_copy(rows_v, agg_sh.at[dstc_v], add=True)
            return 0

        lax.fori_loop(0, nch, main_body, 0)

        @pl.when(cid == 0)
        def _():
            woff = pl.multiple_of(sid * WSTRIDE + sec_base, 8)
            pltpu.sync_copy(wrow_v.at[pl.ds(0, nch * CHUNK)],
                            w_out.at[pl.ds(woff, nch * CHUNK)])

    plsc.subcore_barrier()

    _agg_writeback(agg_sh, agg_out, cid, sid)


def _sc_kernel_c(srce_hbm, dste_hbm, et_hbm, h_hbm, w_hbm, agg_out,
                 agg_sh, src_v, dst_v, et_v, g2_v, dstc_v, wrow_v, rows_v):
    cid = lax.axis_index("c")
    sid = lax.axis_index("s")
    ebase = pl.multiple_of(cid * (E // NC) + sid * EPW_C, 8)

    _zero_fill_2d(rows_v, CHUNK, L)
    _agg_zero(agg_sh, rows_v, sid)

    _fill_tail_i32(src_v, EPW_C, PAD_C - EPW_C, 0)
    _fill_tail_i32(dst_v, EPW_C, PAD_C - EPW_C, N)
    _fill_tail_i32(et_v, EPW_C, PAD_C - EPW_C, 0)
    z = jnp.zeros((LN,), jnp.float32)
    for k in range((PAD_C + LN - EPW_C) // LN):
        wrow_v[pl.ds(EPW_C + k * LN, LN)] = z
    pltpu.sync_copy(srce_hbm.at[pl.ds(ebase, EPW_C)], src_v.at[pl.ds(0, EPW_C)])
    pltpu.sync_copy(dste_hbm.at[pl.ds(ebase, EPW_C)], dst_v.at[pl.ds(0, EPW_C)])
    pltpu.sync_copy(et_hbm.at[pl.ds(ebase, EPW_C)], et_v.at[pl.ds(0, EPW_C)])
    wid = cid * NS + sid
    woff = pl.multiple_of((wid // 2) * WSTRIDE + (wid % 2) * EPW_C, 8)
    pltpu.sync_copy(w_hbm.at[pl.ds(woff, EPW_C)], wrow_v.at[pl.ds(0, EPW_C)])

    plsc.subcore_barrier()

    def main_body(g, _):
        for i in range(CHUNK // LN):
            sl = pl.ds(g * CHUNK + i * LN, LN)
            s16 = src_v[sl]
            t16 = et_v[sl]
            g2_v[pl.ds(i * LN, LN)] = t16 * N + s16
            dstc_v[pl.ds(i * LN, LN)] = dst_v[sl]
        pltpu.sync_copy(h_hbm.at[g2_v], rows_v)

        def scale_body(j, _):
            wj = wrow_v[pl.ds(g * CHUNK + j, LN)][0]
            for hh in range(L // LN):
                sl2 = pl.ds(hh * LN, LN)
                rows_v[j, sl2] = rows_v[j, sl2] * wj
            return 0

        lax.fori_loop(0, CHUNK, scale_body, 0)
        pltpu.sync_copy(rows_v, agg_sh.at[dstc_v], add=True)
        return 0

    lax.fori_loop(0, NCH_C, main_body, 0)

    plsc.subcore_barrier()

    _agg_writeback(agg_sh, agg_out, cid, sid)


_sc_call_a = functools.partial(
    pl.kernel,
    out_type=(jax.ShapeDtypeStruct((NC, N, HH), jnp.float32),
              jax.ShapeDtypeStruct((NS * WSTRIDE,), jnp.float32)),
    mesh=_mesh,
    scratch_types=[
        pltpu.VMEM_SHARED((N * R + LN,), jnp.float32),
        pltpu.VMEM_SHARED((N + 8, HH), jnp.float32),
        pltpu.VMEM((SEC,), jnp.int32),
        pltpu.VMEM((SEC,), jnp.int32),
        pltpu.VMEM((SEC,), jnp.int32),
        pltpu.VMEM((CHUNK,), jnp.int32),
        pltpu.VMEM((CHUNK,), jnp.int32),
        pltpu.VMEM((CHUNK,), jnp.int32),
        pltpu.VMEM((CHUNK,), jnp.float32),
        pltpu.VMEM((CHUNK + LN,), jnp.float32),
        pltpu.VMEM((SEC,), jnp.float32),
        pltpu.VMEM((CHUNK, HH), jnp.float32),
        pltpu.VMEM((2000,), jnp.float32),
    ],
)(_sc_kernel_a)

_sc_call_c = functools.partial(
    pl.kernel,
    out_type=jax.ShapeDtypeStruct((NC, N, L), jnp.float32),
    mesh=_mesh,
    scratch_types=[
        pltpu.VMEM_SHARED((N + 8, L), jnp.float32),
        pltpu.VMEM((PAD_C,), jnp.int32),
        pltpu.VMEM((PAD_C,), jnp.int32),
        pltpu.VMEM((PAD_C,), jnp.int32),
        pltpu.VMEM((CHUNK,), jnp.int32),
        pltpu.VMEM((CHUNK,), jnp.int32),
        pltpu.VMEM((PAD_C + LN,), jnp.float32),
        pltpu.VMEM((CHUNK, L), jnp.float32),
    ],
)(_sc_kernel_c)


BN = 1000


def _tc_mm_body(agg_ref, root1_ref, bias1_ref, wcat_ref, out_ref):
    c = pl.program_id(2)
    x = agg_ref[0] + root1_ref[...] + bias1_ref[...]
    x = jnp.maximum(x, 0.0)
    part = jnp.dot(x, wcat_ref[0], preferred_element_type=jnp.float32,
                   precision=lax.Precision.HIGHEST)

    @pl.when(c == 0)
    def _():
        out_ref[0] = part

    @pl.when(c != 0)
    def _():
        out_ref[0] += part


_tc_mm = pl.pallas_call(
    _tc_mm_body,
    grid=(R + 1, N // BN, NC),
    in_specs=[
        pl.BlockSpec((1, BN, HH), lambda r, n, c: (c, n, 0)),
        pl.BlockSpec((BN, HH), lambda r, n, c: (n, c)),
        pl.BlockSpec((1, HH), lambda r, n, c: (0, c)),
        pl.BlockSpec((1, HH, L), lambda r, n, c: (r, c, 0)),
    ],
    out_specs=pl.BlockSpec((1, BN, L), lambda r, n, c: (r, n, 0)),
    out_shape=jax.ShapeDtypeStruct((R + 1, N, L), jnp.float32),
)


def _tc_fin_body(agg2_ref, x_ref, root2_ref, bias2_ref, out_ref):
    p = jnp.dot(x_ref[...], root2_ref[...], preferred_element_type=jnp.float32,
                precision=lax.Precision.HIGHEST)
    s = agg2_ref[...] + p + bias2_ref[...]
    out_ref[...] = 1.0 / (1.0 + jnp.exp(-s))


_tc_fin = pl.pallas_call(
    _tc_fin_body,
    grid=(N // BN,),
    in_specs=[
        pl.BlockSpec((BN, L), lambda n: (n, 0)),
        pl.BlockSpec((BN, H), lambda n: (n, 0)),
        pl.BlockSpec((H, L), lambda n: (0, 0)),
        pl.BlockSpec((1, L), lambda n: (0, 0)),
    ],
    out_specs=pl.BlockSpec((BN, L), lambda n: (n, 0)),
    out_shape=jax.ShapeDtypeStruct((N, L), jnp.float32),
)


def kernel(edge_index, edge_type, weight1, root1, bias1, weight2, root2, bias2):
    w1_flat = weight1.reshape(R * N * 2, HH)
    src_e = edge_index[0]
    dst_e = edge_index[1]
    agg1, w_edge = _sc_call_a(src_e, dst_e, edge_type, w1_flat)
    x = jnp.concatenate([agg1[0], agg1[1]], axis=1) + root1 + bias1[None, :]
    x = jax.nn.relu(x)
    h_flat = jnp.einsum('nh,rhl->rnl', x, weight2).reshape(R * N, L)
    e_ar = jnp.arange(E, dtype=jnp.int32)
    w = w_edge[(e_ar // EPS_A) * WSTRIDE + e_ar % EPS_A]
    msg2 = h_flat[edge_type * N + src_e] * w[:, None]
    agg2 = jnp.zeros((N, L), jnp.float32).at[dst_e].add(msg2)
    out = _tc_fin(agg2, x, root2, bias2.reshape(1, L))
    return out
